# Initial kernel scaffold; baseline (speedup 1.0000x reference)
import jax
import jax.numpy as jnp
from jax.experimental import pallas as pl


def kernel(image_vectors, image_index, memory_bank):
    raise NotImplementedError("write your pallas kernel here")


# speedup vs baseline: 12.6172x; 12.6172x over previous
import functools

import jax
import jax.numpy as jnp
from jax import lax
from jax.experimental import pallas as pl
from jax.experimental.pallas import tpu as pltpu
from jax.experimental.pallas import tpu_sc as plsc

B = 1024
H = 128
N_TOT = 4097
NUM_IMG = 100000
TEMP = 0.07
PAD_COLS = 100352
TOTAL = B * N_TOT
IDX_ROWS = 33280
PAD_CNT = IDX_ROWS * 128 - TOTAL
SAFE_IDX = PAD_COLS - 1

_info = plsc.get_sparse_core_info()
NC, NS = _info.num_cores, _info.num_subcores
NW = NC * NS
ROWS_PER_W = IDX_ROWS // NW
CHUNK_ROWS = 16
N_CHUNKS = ROWS_PER_W // CHUNK_ROWS


def _mm_body(iv_ref, mb_ref, s_ref):
    s_ref[...] = lax.dot_general(
        iv_ref[...], mb_ref[...],
        dimension_numbers=(((1,), (1,)), ((), ())),
        preferred_element_type=jnp.float32)


_MM_NBLK = 1024


def _matmul(iv, mb_pad):
    return pl.pallas_call(
        _mm_body,
        grid=(PAD_COLS // _MM_NBLK,),
        in_specs=[pl.BlockSpec((B, H), lambda i: (0, 0)),
                  pl.BlockSpec((_MM_NBLK, H), lambda i: (i, 0))],
        out_specs=pl.BlockSpec((B, _MM_NBLK), lambda i: (0, i)),
        out_shape=jax.ShapeDtypeStruct((B, PAD_COLS), jnp.float32),
    )(iv, mb_pad)


_INV_T = 1.0 / TEMP

_sc_mesh = plsc.VectorSubcoreMesh(core_axis_name="c", subcore_axis_name="s")


@functools.partial(
    pl.kernel,
    mesh=_sc_mesh,
    out_type=[jax.ShapeDtypeStruct((IDX_ROWS, 128), jnp.float32),
              jax.ShapeDtypeStruct((NW, 16), jnp.float32)],
    scratch_types=[pltpu.VMEM((CHUNK_ROWS, 128), jnp.int32),
                   pltpu.VMEM((CHUNK_ROWS, 128), jnp.float32),
                   pltpu.VMEM((CHUNK_ROWS, 128), jnp.float32),
                   pltpu.VMEM((16,), jnp.float32),
                   pltpu.SemaphoreType.DMA],
)
def _sc_gather_exp(s_hbm, idx_hbm, probs_hbm, part_hbm,
                   idx_v, rows_v, prob_v, acc_v, sem):
    wid = lax.axis_index("s") * NC + lax.axis_index("c")
    base = wid * ROWS_PER_W

    def chunk(c, acc):
        row0 = base + c * CHUNK_ROWS
        pltpu.sync_copy(idx_hbm.at[pl.ds(row0, CHUNK_ROWS)], idx_v)
        descs = [pltpu.async_copy(s_hbm.at[idx_v.at[j]], rows_v.at[j], sem)
                 for j in range(CHUNK_ROWS)]
        for d in descs:
            d.wait()
        for j in range(CHUNK_ROWS):
            for k in range(8):
                v = rows_v[j, pl.ds(k * 16, 16)]
                p = jnp.exp(v * _INV_T)
                acc = acc + p
                prob_v[j, pl.ds(k * 16, 16)] = p
        pltpu.sync_copy(prob_v, probs_hbm.at[pl.ds(row0, CHUNK_ROWS)])
        return acc

    acc = lax.fori_loop(0, N_CHUNKS, chunk, jnp.zeros((16,), jnp.float32))
    acc_v[...] = acc
    pltpu.sync_copy(acc_v, part_hbm.at[wid])


def _scale_body(part_ref, probs_ref, out_ref):
    s = jnp.sum(part_ref[...]) - float(PAD_CNT)
    z = s / float(TOTAL) * float(NUM_IMG)
    out_ref[...] = probs_ref[...] / z


_SC_RBLK = 4160


def _normalize(partials, probs_pad):
    return pl.pallas_call(
        _scale_body,
        grid=(IDX_ROWS // _SC_RBLK,),
        in_specs=[pl.BlockSpec((NW, 16), lambda i: (0, 0)),
                  pl.BlockSpec((_SC_RBLK, 128), lambda i: (i, 0))],
        out_specs=pl.BlockSpec((_SC_RBLK, 128), lambda i: (i, 0)),
        out_shape=jax.ShapeDtypeStruct((IDX_ROWS, 128), jnp.float32),
    )(partials, probs_pad)


def kernel(image_vectors, image_index, memory_bank):
    skey = jax.random.key(12345)
    samp = jax.random.randint(skey, (B, N_TOT), 0, NUM_IMG, dtype=jnp.int32)
    samp = samp.at[:, 0].set(image_index)
    flat_idx = (jnp.arange(B, dtype=jnp.int32)[:, None] * PAD_COLS
                + samp).reshape(-1)
    flat_idx = jnp.concatenate(
        [flat_idx, jnp.full((PAD_CNT,), SAFE_IDX, jnp.int32)])
    idx2d = flat_idx.reshape(IDX_ROWS, 128)

    mb_pad = jnp.pad(memory_bank, ((0, PAD_COLS - NUM_IMG), (0, 0)))
    s = _matmul(image_vectors, mb_pad)
    probs_pad, partials = _sc_gather_exp(s.reshape(-1), idx2d)
    out_pad = _normalize(partials, probs_pad)
    return out_pad.reshape(-1)[:TOTAL].reshape(B, N_TOT)


# speedup vs baseline: 13.9843x; 1.1083x over previous
import functools

import jax
import jax.numpy as jnp
from jax import lax
from jax.experimental import pallas as pl
from jax.experimental.pallas import tpu as pltpu
from jax.experimental.pallas import tpu_sc as plsc

B = 1024
H = 128
N_TOT = 4097
NUM_IMG = 100000
TEMP = 0.07
PAD_COLS = 100352
TOTAL = B * N_TOT
IDX_ROWS = 33280
PAD_CNT = IDX_ROWS * 128 - TOTAL
SAFE_IDX = PAD_COLS - 1

_info = plsc.get_sparse_core_info()
NC, NS = _info.num_cores, _info.num_subcores
NW = NC * NS
ROWS_PER_W = IDX_ROWS // NW
CHUNK_ROWS = 16
N_CHUNKS = ROWS_PER_W // CHUNK_ROWS


_MM_NBLK = 1024


def _mm_body(iv_ref, mb_ref, s_ref):
    t = lax.dot_general(
        iv_ref[...], mb_ref[...],
        dimension_numbers=(((1,), (1,)), ((), ())),
        preferred_element_type=jnp.float32)
    s_ref[...] = t.reshape(B, _MM_NBLK // 128, 128)


def _matmul(iv, mb_pad):
    return pl.pallas_call(
        _mm_body,
        grid=(PAD_COLS // _MM_NBLK,),
        in_specs=[pl.BlockSpec((B, H), lambda i: (0, 0)),
                  pl.BlockSpec((_MM_NBLK, H), lambda i: (i, 0))],
        out_specs=pl.BlockSpec((B, _MM_NBLK // 128, 128),
                               lambda i: (0, i, 0)),
        out_shape=jax.ShapeDtypeStruct((B, PAD_COLS // 128, 128),
                                       jnp.float32),
    )(iv, mb_pad)


_INV_T = 1.0 / TEMP

_sc_mesh = plsc.VectorSubcoreMesh(core_axis_name="c", subcore_axis_name="s")


@functools.partial(
    pl.kernel,
    mesh=_sc_mesh,
    out_type=[jax.ShapeDtypeStruct((IDX_ROWS, 128), jnp.float32),
              jax.ShapeDtypeStruct((NW, 16), jnp.float32)],
    scratch_types=[pltpu.VMEM((CHUNK_ROWS, 128), jnp.int32),
                   pltpu.VMEM((CHUNK_ROWS, 128), jnp.float32),
                   pltpu.VMEM((CHUNK_ROWS, 128), jnp.float32),
                   pltpu.VMEM((16,), jnp.float32),
                   pltpu.SemaphoreType.DMA],
)
def _sc_gather_exp(s_hbm, idx_hbm, probs_hbm, part_hbm,
                   idx_v, rows_v, prob_v, acc_v, sem):
    wid = lax.axis_index("s") * NC + lax.axis_index("c")
    base = wid * ROWS_PER_W

    def chunk(c, acc):
        row0 = base + c * CHUNK_ROWS
        pltpu.sync_copy(idx_hbm.at[pl.ds(row0, CHUNK_ROWS)], idx_v)
        descs = [pltpu.async_copy(s_hbm.at[idx_v.at[j]], rows_v.at[j], sem)
                 for j in range(CHUNK_ROWS)]
        for d in descs:
            d.wait()
        for j in range(CHUNK_ROWS):
            for k in range(8):
                v = rows_v[j, pl.ds(k * 16, 16)]
                p = jnp.exp(v * _INV_T)
                acc = acc + p
                prob_v[j, pl.ds(k * 16, 16)] = p
        pltpu.sync_copy(prob_v, probs_hbm.at[pl.ds(row0, CHUNK_ROWS)])
        return acc

    acc = lax.fori_loop(0, N_CHUNKS, chunk, jnp.zeros((16,), jnp.float32))
    acc_v[...] = acc
    pltpu.sync_copy(acc_v, part_hbm.at[wid])


def _scale_body(part_ref, probs_ref, out_ref):
    s = jnp.sum(part_ref[...]) - float(PAD_CNT)
    z = s / float(TOTAL) * float(NUM_IMG)
    out_ref[...] = probs_ref[...] / z


_SC_RBLK = 4160


def _normalize(partials, probs_pad):
    return pl.pallas_call(
        _scale_body,
        grid=(IDX_ROWS // _SC_RBLK,),
        in_specs=[pl.BlockSpec((NW, 16), lambda i: (0, 0)),
                  pl.BlockSpec((_SC_RBLK, 128), lambda i: (i, 0))],
        out_specs=pl.BlockSpec((_SC_RBLK, 128), lambda i: (i, 0)),
        out_shape=jax.ShapeDtypeStruct((IDX_ROWS, 128), jnp.float32),
    )(partials, probs_pad)


def kernel(image_vectors, image_index, memory_bank):
    skey = jax.random.key(12345)
    samp = jax.random.randint(skey, (B, N_TOT), 0, NUM_IMG, dtype=jnp.int32)
    samp = samp.at[:, 0].set(image_index)
    flat_idx = (jnp.arange(B, dtype=jnp.int32)[:, None] * PAD_COLS
                + samp).reshape(-1)
    flat_idx = jnp.concatenate(
        [flat_idx, jnp.full((PAD_CNT,), SAFE_IDX, jnp.int32)])
    idx2d = flat_idx.reshape(IDX_ROWS, 128)

    mb_pad = jnp.pad(memory_bank, ((0, PAD_COLS - NUM_IMG), (0, 0)))
    s = _matmul(image_vectors, mb_pad)
    probs_pad, partials = _sc_gather_exp(s.reshape(-1), idx2d)
    out_pad = _normalize(partials, probs_pad)
    return out_pad.reshape(-1)[:TOTAL].reshape(B, N_TOT)


# speedup vs baseline: 14.7309x; 1.0534x over previous
import functools

import jax
import jax.numpy as jnp
from jax import lax
from jax.experimental import pallas as pl
from jax.experimental.pallas import tpu as pltpu
from jax.experimental.pallas import tpu_sc as plsc

B = 1024
H = 128
N_TOT = 4097
NUM_IMG = 100000
TEMP = 0.07
PAD_COLS = 100352
TOTAL = B * N_TOT
IDX_ROWS = 33280
PAD_CNT = IDX_ROWS * 128 - TOTAL
SAFE_IDX = PAD_COLS - 1

_info = plsc.get_sparse_core_info()
NC, NS = _info.num_cores, _info.num_subcores
NW = NC * NS
ROWS_PER_W = IDX_ROWS // NW
CHUNK_ROWS = 16
N_CHUNKS = ROWS_PER_W // CHUNK_ROWS


_MM_NBLK = 1024


def _mm_body(iv_ref, mb_ref, s_ref):
    t = lax.dot_general(
        iv_ref[...], mb_ref[...],
        dimension_numbers=(((1,), (1,)), ((), ())),
        preferred_element_type=jnp.float32)
    s_ref[...] = t.reshape(B, _MM_NBLK // 128, 128)


def _matmul(iv, mb_pad):
    return pl.pallas_call(
        _mm_body,
        grid=(PAD_COLS // _MM_NBLK,),
        in_specs=[pl.BlockSpec((B, H), lambda i: (0, 0)),
                  pl.BlockSpec((_MM_NBLK, H), lambda i: (i, 0))],
        out_specs=pl.BlockSpec((B, _MM_NBLK // 128, 128),
                               lambda i: (0, i, 0)),
        out_shape=jax.ShapeDtypeStruct((B, PAD_COLS // 128, 128),
                                       jnp.float32),
    )(iv, mb_pad)


_INV_T = 1.0 / TEMP

_sc_mesh = plsc.VectorSubcoreMesh(core_axis_name="c", subcore_axis_name="s")


@functools.partial(
    pl.kernel,
    mesh=_sc_mesh,
    out_type=[jax.ShapeDtypeStruct((IDX_ROWS, 128), jnp.float32),
              jax.ShapeDtypeStruct((NW, 16), jnp.float32)],
    scratch_types=[pltpu.VMEM((CHUNK_ROWS, 128), jnp.int32),
                   pltpu.VMEM((CHUNK_ROWS, 128), jnp.float32),
                   pltpu.VMEM((CHUNK_ROWS, 128), jnp.float32),
                   pltpu.VMEM((16,), jnp.float32),
                   pltpu.SemaphoreType.DMA],
)
def _sc_gather_exp(s_hbm, idx_hbm, probs_hbm, part_hbm,
                   idx_v, rows_v, prob_v, acc_v, sem):
    cid = lax.axis_index("c")
    sid = lax.axis_index("s")
    r_fast, r_slow = 1360, 720
    base = jnp.where(cid == 0, sid * r_fast,
                     NS * r_fast + sid * r_slow)
    n_chunks = jnp.where(cid == 0, r_fast // CHUNK_ROWS,
                         r_slow // CHUNK_ROWS)

    def chunk(c, acc):
        row0 = base + c * CHUNK_ROWS
        pltpu.sync_copy(idx_hbm.at[pl.ds(row0, CHUNK_ROWS)], idx_v)
        descs = [pltpu.async_copy(s_hbm.at[idx_v.at[j]], rows_v.at[j], sem)
                 for j in range(CHUNK_ROWS)]
        for d in descs:
            d.wait()
        for j in range(CHUNK_ROWS):
            for k in range(8):
                v = rows_v[j, pl.ds(k * 16, 16)]
                p = jnp.exp(v * _INV_T)
                acc = acc + p
                prob_v[j, pl.ds(k * 16, 16)] = p
        pltpu.sync_copy(prob_v, probs_hbm.at[pl.ds(row0, CHUNK_ROWS)])
        return acc

    acc = lax.fori_loop(0, n_chunks, chunk, jnp.zeros((16,), jnp.float32))
    acc_v[...] = acc
    pltpu.sync_copy(acc_v, part_hbm.at[sid * NC + cid])


def _scale_body(part_ref, probs_ref, out_ref):
    s = jnp.sum(part_ref[...]) - float(PAD_CNT)
    z = s / float(TOTAL) * float(NUM_IMG)
    out_ref[...] = probs_ref[...] / z


_SC_RBLK = 4160


def _normalize(partials, probs_pad):
    return pl.pallas_call(
        _scale_body,
        grid=(IDX_ROWS // _SC_RBLK,),
        in_specs=[pl.BlockSpec((NW, 16), lambda i: (0, 0)),
                  pl.BlockSpec((_SC_RBLK, 128), lambda i: (i, 0))],
        out_specs=pl.BlockSpec((_SC_RBLK, 128), lambda i: (i, 0)),
        out_shape=jax.ShapeDtypeStruct((IDX_ROWS, 128), jnp.float32),
    )(partials, probs_pad)


def kernel(image_vectors, image_index, memory_bank):
    skey = jax.random.key(12345)
    samp = jax.random.randint(skey, (B, N_TOT), 0, NUM_IMG, dtype=jnp.int32)
    samp = samp.at[:, 0].set(image_index)
    flat_idx = (jnp.arange(B, dtype=jnp.int32)[:, None] * PAD_COLS
                + samp).reshape(-1)
    flat_idx = jnp.concatenate(
        [flat_idx, jnp.full((PAD_CNT,), SAFE_IDX, jnp.int32)])
    idx2d = flat_idx.reshape(IDX_ROWS, 128)

    mb_pad = jnp.pad(memory_bank, ((0, PAD_COLS - NUM_IMG), (0, 0)))
    s = _matmul(image_vectors, mb_pad)
    probs_pad, partials = _sc_gather_exp(s.reshape(-1), idx2d)
    out_pad = _normalize(partials, probs_pad)
    return out_pad.reshape(-1)[:TOTAL].reshape(B, N_TOT)
